# Initial kernel scaffold; baseline (speedup 1.0000x reference)
import jax
import jax.numpy as jnp
from jax.experimental import pallas as pl


def kernel(x_, edge_index, edge_weight, W, b):
    raise NotImplementedError("write your pallas kernel here")


# speedup vs baseline: 12.8009x; 12.8009x over previous
import jax
import jax.numpy as jnp
from jax import lax
from jax.experimental import pallas as pl
from jax.experimental.pallas import tpu as pltpu
from jax.experimental.pallas import tpu_sc as plsc

D = 128
N_PAD = 10240
ROWS_PER_TILE = 640
E_PAD = 327680
EDGES_PER_TILE = E_PAD // 32
CHUNK = 256
N_CHUNKS = EDGES_PER_TILE // CHUNK


def _dense_body(x_ref, w_ref, b_ref, h_ref):
    acc = lax.dot_general(x_ref[...], w_ref[...], (((1,), (1,)), ((), ())),
                          preferred_element_type=jnp.float32)
    h_ref[...] = jnp.maximum(acc + b_ref[...], 0.0)


def _dense(x, W, b):
    m = x.shape[0]
    bm = 1000
    return pl.pallas_call(
        _dense_body,
        grid=(m // bm,),
        in_specs=[pl.BlockSpec((bm, D), lambda i: (i, 0)),
                  pl.BlockSpec((D, D), lambda i: (0, 0)),
                  pl.BlockSpec((1, D), lambda i: (0, 0))],
        out_specs=pl.BlockSpec((bm, D), lambda i: (i, 0)),
        out_shape=jax.ShapeDtypeStruct((m, D), jnp.float32),
    )(x, W, b.reshape(1, D))


def _sc_body(h_hbm, src_hbm, dst_hbm, w_hbm, outp_hbm, degp_hbm,
             src_v, dst_v, w_v, rows_v, zb_v, accum_sh, deg_sh):
    c = lax.axis_index("c")
    s = lax.axis_index("s")
    wid = c * 16 + s

    zeros16 = jnp.zeros((16,), jnp.float32)

    @pl.loop(0, CHUNK)
    def _(r):
        for j in range(8):
            rows_v[r, pl.ds(16 * j, 16)] = zeros16

    @pl.loop(0, ROWS_PER_TILE // 16)
    def _(i):
        zb_v[pl.ds(pl.multiple_of(i * 16, 16), 16)] = zeros16

    base_row = pl.multiple_of(s * ROWS_PER_TILE, ROWS_PER_TILE)
    pltpu.sync_copy(rows_v, accum_sh.at[pl.ds(base_row, CHUNK)])
    pltpu.sync_copy(rows_v, accum_sh.at[pl.ds(base_row + CHUNK, CHUNK)])
    pltpu.sync_copy(rows_v.at[pl.ds(0, 128)],
                    accum_sh.at[pl.ds(base_row + 2 * CHUNK, 128)])
    pltpu.sync_copy(zb_v, deg_sh.at[pl.ds(base_row, ROWS_PER_TILE)])
    plsc.subcore_barrier()

    tile_idx_row0 = wid * (EDGES_PER_TILE // 128)
    tile_edge0 = wid * EDGES_PER_TILE

    @pl.loop(0, N_CHUNKS)
    def _(k):
        r0 = pl.multiple_of(tile_idx_row0 + k * 2, 2)
        e0 = pl.multiple_of(tile_edge0 + k * CHUNK, CHUNK)
        pltpu.sync_copy(src_hbm.at[pl.ds(r0, 2)], src_v)
        pltpu.sync_copy(dst_hbm.at[pl.ds(r0, 2)], dst_v)
        pltpu.sync_copy(w_hbm.at[pl.ds(e0, CHUNK)], w_v)
        for j in range(2):
            pltpu.sync_copy(h_hbm.at[dst_v.at[j]],
                            rows_v.at[pl.ds(128 * j, 128)])

        @pl.loop(0, CHUNK // 16)
        def _(g):
            w16 = w_v[pl.ds(g * 16, 16)]
            e0 = g * 16
            for t in range(16):
                wb = jnp.broadcast_to(w16[t], (16,))
                for j in range(8):
                    sl = pl.ds(16 * j, 16)
                    rows_v[e0 + t, sl] = rows_v[e0 + t, sl] * wb

        for j in range(2):
            pltpu.sync_copy(rows_v.at[pl.ds(128 * j, 128)],
                            accum_sh.at[src_v.at[j]], add=True)
            pltpu.sync_copy(w_v.at[pl.ds(128 * j, 128)],
                            deg_sh.at[src_v.at[j]], add=True)

    plsc.subcore_barrier()

    pltpu.sync_copy(accum_sh.at[pl.ds(base_row, ROWS_PER_TILE)],
                    outp_hbm.at[c].at[pl.ds(base_row, ROWS_PER_TILE)])
    pltpu.sync_copy(deg_sh.at[pl.ds(base_row, ROWS_PER_TILE)],
                    degp_hbm.at[c].at[pl.ds(base_row, ROWS_PER_TILE)])


def _sparse(h, src2d, dst2d, w1d):
    fn = pl.kernel(
        _sc_body,
        out_type=[jax.ShapeDtypeStruct((2, N_PAD, D), jnp.float32),
                  jax.ShapeDtypeStruct((2, N_PAD), jnp.float32)],
        mesh=plsc.VectorSubcoreMesh(core_axis_name="c", subcore_axis_name="s"),
        scratch_types=[
            pltpu.VMEM((2, 128), jnp.int32),
            pltpu.VMEM((2, 128), jnp.int32),
            pltpu.VMEM((CHUNK,), jnp.float32),
            pltpu.VMEM((CHUNK, D), jnp.float32),
            pltpu.VMEM((ROWS_PER_TILE,), jnp.float32),
            pltpu.VMEM_SHARED((N_PAD, D), jnp.float32),
            pltpu.VMEM_SHARED((N_PAD,), jnp.float32),
        ],
    )
    return fn(h, src2d, dst2d, w1d)


def _epilogue_body(p0_ref, p1_ref, d0_ref, d1_ref, o_ref):
    d = d0_ref[...] + d1_ref[...]
    d = jnp.where(d < 0.5, d + 1.0, d)
    o_ref[...] = (p0_ref[...] + p1_ref[...]) * (1.0 / d)


def _epilogue(p0, p1, d0, d1):
    n = p0.shape[0]
    return pl.pallas_call(
        _epilogue_body,
        out_shape=jax.ShapeDtypeStruct((n, D), jnp.float32),
    )(p0, p1, d0, d1)


def kernel(x_, edge_index, edge_weight, W, b):
    n = x_.shape[0]
    e = edge_index.shape[1]
    h = _dense(x_, W, b)

    src = edge_index[0]
    dst = edge_index[1]
    pad = E_PAD - e
    pad_idx = (jnp.arange(pad, dtype=jnp.int32) * 37) % n
    src_p = jnp.concatenate([src, pad_idx]).reshape(E_PAD // 128, 128)
    dst_p = jnp.concatenate([dst, pad_idx]).reshape(E_PAD // 128, 128)
    w_p = jnp.concatenate([edge_weight, jnp.zeros((pad,), jnp.float32)])

    outp, degp = _sparse(h, src_p, dst_p, w_p)
    return _epilogue(outp[0, :n], outp[1, :n],
                     degp[0, :n, None], degp[1, :n, None])


# speedup vs baseline: 19.3554x; 1.5120x over previous
import jax
import jax.numpy as jnp
from jax import lax
from jax.experimental import pallas as pl
from jax.experimental.pallas import tpu as pltpu
from jax.experimental.pallas import tpu_sc as plsc

D = 128
N_PAD = 10240
ROWS_PER_TILE = 640
E_PAD = 327680
EDGES_PER_TILE = E_PAD // 32
CHUNK = 128
N_CHUNKS = EDGES_PER_TILE // CHUNK


def _dense_body(x_ref, w_ref, b_ref, h_ref):
    acc = lax.dot_general(x_ref[...], w_ref[...], (((1,), (1,)), ((), ())),
                          preferred_element_type=jnp.float32)
    h_ref[...] = jnp.maximum(acc + b_ref[...], 0.0)


def _dense(x, W, b):
    m = x.shape[0]
    bm = 1000
    return pl.pallas_call(
        _dense_body,
        grid=(m // bm,),
        in_specs=[pl.BlockSpec((bm, D), lambda i: (i, 0)),
                  pl.BlockSpec((D, D), lambda i: (0, 0)),
                  pl.BlockSpec((1, D), lambda i: (0, 0))],
        out_specs=pl.BlockSpec((bm, D), lambda i: (i, 0)),
        out_shape=jax.ShapeDtypeStruct((m, D), jnp.float32),
    )(x, W, b.reshape(1, D))


def _sc_body(h_hbm, idx_hbm, w_hbm, outp_hbm, degp_hbm,
             iv0, iv1, w0, w1, rows0, rows1, zb_v,
             sg0, sg1, accum_sh, deg_sh):
    c = lax.axis_index("c")
    s = lax.axis_index("s")
    wid = c * 16 + s

    ivs = (iv0, iv1)
    ws = (w0, w1)
    rows = (rows0, rows1)
    sgs = (sg0, sg1)

    zeros16 = jnp.zeros((16,), jnp.float32)

    @pl.loop(0, CHUNK)
    def _(r):
        for j in range(8):
            rows0[r, pl.ds(16 * j, 16)] = zeros16

    @pl.loop(0, ROWS_PER_TILE // 16)
    def _(i):
        zb_v[pl.ds(pl.multiple_of(i * 16, 16), 16)] = zeros16

    base_row = pl.multiple_of(s * ROWS_PER_TILE, ROWS_PER_TILE)
    for q in range(ROWS_PER_TILE // CHUNK):
        pltpu.sync_copy(rows0, accum_sh.at[pl.ds(base_row + q * CHUNK, CHUNK)])
    pltpu.sync_copy(zb_v, deg_sh.at[pl.ds(base_row, ROWS_PER_TILE)])
    plsc.subcore_barrier()

    chunk0 = wid * N_CHUNKS
    tile_edge0 = wid * EDGES_PER_TILE

    def start(b, k):
        pltpu.sync_copy(idx_hbm.at[chunk0 + k], ivs[b])
        e0 = pl.multiple_of(tile_edge0 + k * CHUNK, CHUNK)
        pltpu.async_copy(w_hbm.at[pl.ds(e0, CHUNK)], ws[b], sgs[b])
        pltpu.async_copy(h_hbm.at[ivs[b].at[1]], rows[b], sgs[b])

    def drain(b):
        pltpu.make_async_copy(w_hbm.at[pl.ds(0, CHUNK)], ws[b], sgs[b]).wait()
        pltpu.make_async_copy(h_hbm.at[pl.ds(0, CHUNK)], rows[b], sgs[b]).wait()

    def scale(b):
        @pl.loop(0, CHUNK // 16)
        def _(g):
            w16 = ws[b][pl.ds(g * 16, 16)]
            e0 = g * 16
            for t in range(16):
                wb = jnp.broadcast_to(w16[t], (16,))
                for j in range(8):
                    sl = pl.ds(16 * j, 16)
                    rows[b][e0 + t, sl] = rows[b][e0 + t, sl] * wb

    def scatter(b):
        pltpu.sync_copy(rows[b], accum_sh.at[ivs[b].at[0]], add=True)
        pltpu.sync_copy(ws[b], deg_sh.at[ivs[b].at[0]], add=True)

    start(0, 0)
    start(1, 1)

    @pl.loop(0, N_CHUNKS // 2 - 1)
    def _(i):
        k = i * 2
        for b in range(2):
            drain(b)
            scale(b)
            scatter(b)
            start(b, k + 2 + b)

    for b in range(2):
        drain(b)
        scale(b)
        scatter(b)

    plsc.subcore_barrier()

    pltpu.sync_copy(accum_sh.at[pl.ds(base_row, ROWS_PER_TILE)],
                    outp_hbm.at[c].at[pl.ds(base_row, ROWS_PER_TILE)])
    pltpu.sync_copy(deg_sh.at[pl.ds(base_row, ROWS_PER_TILE)],
                    degp_hbm.at[c].at[pl.ds(base_row, ROWS_PER_TILE)])


def _sparse(h, idx_all, w1d):
    fn = pl.kernel(
        _sc_body,
        out_type=[jax.ShapeDtypeStruct((2, N_PAD, D), jnp.float32),
                  jax.ShapeDtypeStruct((2, N_PAD), jnp.float32)],
        mesh=plsc.VectorSubcoreMesh(core_axis_name="c", subcore_axis_name="s"),
        scratch_types=[
            pltpu.VMEM((2, 128), jnp.int32),
            pltpu.VMEM((2, 128), jnp.int32),
            pltpu.VMEM((CHUNK,), jnp.float32),
            pltpu.VMEM((CHUNK,), jnp.float32),
            pltpu.VMEM((CHUNK, D), jnp.float32),
            pltpu.VMEM((CHUNK, D), jnp.float32),
            pltpu.VMEM((ROWS_PER_TILE,), jnp.float32),
            pltpu.SemaphoreType.DMA,
            pltpu.SemaphoreType.DMA,
            pltpu.VMEM_SHARED((N_PAD, D), jnp.float32),
            pltpu.VMEM_SHARED((N_PAD,), jnp.float32),
        ],
    )
    return fn(h, idx_all, w1d)


def _epilogue_body(p0_ref, p1_ref, d0_ref, d1_ref, o_ref):
    d = d0_ref[...] + d1_ref[...]
    d = jnp.where(d < 0.5, d + 1.0, d)
    o_ref[...] = (p0_ref[...] + p1_ref[...]) * (1.0 / d)


def _epilogue(p0, p1, d0, d1):
    n = p0.shape[0]
    return pl.pallas_call(
        _epilogue_body,
        out_shape=jax.ShapeDtypeStruct((n, D), jnp.float32),
    )(p0, p1, d0, d1)


def kernel(x_, edge_index, edge_weight, W, b):
    n = x_.shape[0]
    e = edge_index.shape[1]
    h = _dense(x_, W, b)

    src = edge_index[0]
    dst = edge_index[1]
    pad = E_PAD - e
    pad_idx = (jnp.arange(pad, dtype=jnp.int32) * 37) % n
    src_p = jnp.concatenate([src, pad_idx]).reshape(E_PAD // CHUNK, 1, 128)
    dst_p = jnp.concatenate([dst, pad_idx]).reshape(E_PAD // CHUNK, 1, 128)
    idx_all = jnp.concatenate([src_p, dst_p], axis=1)
    w_p = jnp.concatenate([edge_weight, jnp.zeros((pad,), jnp.float32)])

    outp, degp = _sparse(h, idx_all, w_p)
    return _epilogue(outp[0, :n], outp[1, :n],
                     degp[0, :n, None], degp[1, :n, None])


# speedup vs baseline: 19.3591x; 1.0002x over previous
import jax
import jax.numpy as jnp
from jax import lax
from jax.experimental import pallas as pl
from jax.experimental.pallas import tpu as pltpu
from jax.experimental.pallas import tpu_sc as plsc

D = 128
N_PAD = 10240
ROWS_PER_TILE = 640
E_PAD = 327680
EDGES_PER_TILE = E_PAD // 32
CHUNK = 128
N_CHUNKS = EDGES_PER_TILE // CHUNK


def _dense_body(x_ref, w_ref, b_ref, h_ref):
    acc = lax.dot_general(x_ref[...], w_ref[...], (((1,), (1,)), ((), ())),
                          preferred_element_type=jnp.float32)
    h_ref[...] = jnp.maximum(acc + b_ref[...], 0.0)


def _dense(x, W, b):
    m = x.shape[0]
    bm = 1000
    return pl.pallas_call(
        _dense_body,
        grid=(m // bm,),
        in_specs=[pl.BlockSpec((bm, D), lambda i: (i, 0)),
                  pl.BlockSpec((D, D), lambda i: (0, 0)),
                  pl.BlockSpec((1, D), lambda i: (0, 0))],
        out_specs=pl.BlockSpec((bm, D), lambda i: (i, 0)),
        out_shape=jax.ShapeDtypeStruct((m, D), jnp.float32),
    )(x, W, b.reshape(1, D))


def _sc_body(h_hbm, idx_hbm, w_hbm, outp_hbm, degp_hbm,
             iv0, iv1, w0, w1, rows0, rows1, zb_v,
             sg0, sg1, accum_sh, deg_sh):
    c = lax.axis_index("c")
    s = lax.axis_index("s")
    wid = c * 16 + s

    ivs = (iv0, iv1)
    ws = (w0, w1)
    rows = (rows0, rows1)
    sgs = (sg0, sg1)

    zeros16 = jnp.zeros((16,), jnp.float32)

    @pl.loop(0, CHUNK)
    def _(r):
        for j in range(8):
            rows0[r, pl.ds(16 * j, 16)] = zeros16

    @pl.loop(0, ROWS_PER_TILE // 16)
    def _(i):
        zb_v[pl.ds(pl.multiple_of(i * 16, 16), 16)] = zeros16

    base_row = pl.multiple_of(s * ROWS_PER_TILE, ROWS_PER_TILE)
    for q in range(ROWS_PER_TILE // CHUNK):
        pltpu.sync_copy(rows0, accum_sh.at[pl.ds(base_row + q * CHUNK, CHUNK)])
    pltpu.sync_copy(zb_v, deg_sh.at[pl.ds(base_row, ROWS_PER_TILE)])
    plsc.subcore_barrier()

    chunk0 = wid * N_CHUNKS
    tile_edge0 = wid * EDGES_PER_TILE

    def start(b, k):
        pltpu.sync_copy(idx_hbm.at[chunk0 + k], ivs[b])
        e0 = pl.multiple_of(tile_edge0 + k * CHUNK, CHUNK)
        pltpu.async_copy(w_hbm.at[pl.ds(e0, CHUNK)], ws[b], sgs[b])
        pltpu.async_copy(h_hbm.at[ivs[b].at[1]], rows[b], sgs[b])

    def drain(b):
        pltpu.make_async_copy(w_hbm.at[pl.ds(0, CHUNK)], ws[b], sgs[b]).wait()
        pltpu.make_async_copy(h_hbm.at[pl.ds(0, CHUNK)], rows[b], sgs[b]).wait()

    def scale(b):
        @plsc.parallel_loop(0, CHUNK // 16, unroll=2)
        def _(g):
            w16 = ws[b][pl.ds(g * 16, 16)]
            e0 = g * 16
            for t in range(16):
                wb = jnp.broadcast_to(w16[t], (16,))
                for j in range(8):
                    sl = pl.ds(16 * j, 16)
                    rows[b][e0 + t, sl] = rows[b][e0 + t, sl] * wb

    def scatter(b):
        pltpu.sync_copy(rows[b], accum_sh.at[ivs[b].at[0]], add=True)
        pltpu.sync_copy(ws[b], deg_sh.at[ivs[b].at[0]], add=True)

    start(0, 0)
    start(1, 1)

    @pl.loop(0, N_CHUNKS // 2 - 1)
    def _(i):
        k = i * 2
        for b in range(2):
            drain(b)
            scale(b)
            scatter(b)
            start(b, k + 2 + b)

    for b in range(2):
        drain(b)
        scale(b)
        scatter(b)

    plsc.subcore_barrier()

    pltpu.sync_copy(accum_sh.at[pl.ds(base_row, ROWS_PER_TILE)],
                    outp_hbm.at[c].at[pl.ds(base_row, ROWS_PER_TILE)])
    pltpu.sync_copy(deg_sh.at[pl.ds(base_row, ROWS_PER_TILE)],
                    degp_hbm.at[c].at[pl.ds(base_row, ROWS_PER_TILE)])


def _sparse(h, idx_all, w1d):
    fn = pl.kernel(
        _sc_body,
        out_type=[jax.ShapeDtypeStruct((2, N_PAD, D), jnp.float32),
                  jax.ShapeDtypeStruct((2, N_PAD), jnp.float32)],
        mesh=plsc.VectorSubcoreMesh(core_axis_name="c", subcore_axis_name="s"),
        scratch_types=[
            pltpu.VMEM((2, 128), jnp.int32),
            pltpu.VMEM((2, 128), jnp.int32),
            pltpu.VMEM((CHUNK,), jnp.float32),
            pltpu.VMEM((CHUNK,), jnp.float32),
            pltpu.VMEM((CHUNK, D), jnp.float32),
            pltpu.VMEM((CHUNK, D), jnp.float32),
            pltpu.VMEM((ROWS_PER_TILE,), jnp.float32),
            pltpu.SemaphoreType.DMA,
            pltpu.SemaphoreType.DMA,
            pltpu.VMEM_SHARED((N_PAD, D), jnp.float32),
            pltpu.VMEM_SHARED((N_PAD,), jnp.float32),
        ],
    )
    return fn(h, idx_all, w1d)


def _epilogue_body(p0_ref, p1_ref, d0_ref, d1_ref, o_ref):
    d = d0_ref[...] + d1_ref[...]
    d = jnp.where(d < 0.5, d + 1.0, d)
    o_ref[...] = (p0_ref[...] + p1_ref[...]) * (1.0 / d)


def _epilogue(p0, p1, d0, d1):
    n = p0.shape[0]
    return pl.pallas_call(
        _epilogue_body,
        out_shape=jax.ShapeDtypeStruct((n, D), jnp.float32),
    )(p0, p1, d0, d1)


def kernel(x_, edge_index, edge_weight, W, b):
    n = x_.shape[0]
    e = edge_index.shape[1]
    h = _dense(x_, W, b)

    src = edge_index[0]
    dst = edge_index[1]
    pad = E_PAD - e
    pad_idx = (jnp.arange(pad, dtype=jnp.int32) * 37) % n
    src_p = jnp.concatenate([src, pad_idx]).reshape(E_PAD // CHUNK, 1, 128)
    dst_p = jnp.concatenate([dst, pad_idx]).reshape(E_PAD // CHUNK, 1, 128)
    idx_all = jnp.concatenate([src_p, dst_p], axis=1)
    w_p = jnp.concatenate([edge_weight, jnp.zeros((pad,), jnp.float32)])

    outp, degp = _sparse(h, idx_all, w_p)
    return _epilogue(outp[0, :n], outp[1, :n],
                     degp[0, :n, None], degp[1, :n, None])


# speedup vs baseline: 20.1786x; 1.0423x over previous
import jax
import jax.numpy as jnp
from jax import lax
from jax.experimental import pallas as pl
from jax.experimental.pallas import tpu as pltpu
from jax.experimental.pallas import tpu_sc as plsc

D = 128
N_PAD = 10240
ROWS_PER_TILE = 640
E_PAD = 327680
EDGES_PER_TILE = E_PAD // 32
CHUNK = 128
N_CHUNKS = EDGES_PER_TILE // CHUNK


def _dense_body(x_ref, w_ref, b_ref, h_ref):
    acc = lax.dot_general(x_ref[...], w_ref[...], (((1,), (1,)), ((), ())),
                          preferred_element_type=jnp.float32)
    h_ref[...] = jnp.maximum(acc + b_ref[...], 0.0)


def _dense(x, W, b):
    m = x.shape[0]
    bm = 1000
    return pl.pallas_call(
        _dense_body,
        grid=(m // bm,),
        in_specs=[pl.BlockSpec((bm, D), lambda i: (i, 0)),
                  pl.BlockSpec((D, D), lambda i: (0, 0)),
                  pl.BlockSpec((1, D), lambda i: (0, 0))],
        out_specs=pl.BlockSpec((bm, D), lambda i: (i, 0)),
        out_shape=jax.ShapeDtypeStruct((m, D), jnp.float32),
    )(x, W, b.reshape(1, D))


def _sc_body(h_hbm, idx_hbm, w_hbm, outp_hbm, degp_hbm,
             iv00, iv01, iv10, iv11, w0, w1, rows0, rows1, zb_v,
             sg0, sg1, ss0, ss1, accum_sh, deg_sh):
    c = lax.axis_index("c")
    s = lax.axis_index("s")
    wid = c * 16 + s

    ivs = ((iv00, iv01), (iv10, iv11))
    ws = (w0, w1)
    rows = (rows0, rows1)
    sgs = (sg0, sg1)
    sss = (ss0, ss1)

    zeros16 = jnp.zeros((16,), jnp.float32)

    @pl.loop(0, CHUNK)
    def _(r):
        for j in range(8):
            rows0[r, pl.ds(16 * j, 16)] = zeros16

    @pl.loop(0, ROWS_PER_TILE // 16)
    def _(i):
        zb_v[pl.ds(pl.multiple_of(i * 16, 16), 16)] = zeros16

    base_row = pl.multiple_of(s * ROWS_PER_TILE, ROWS_PER_TILE)
    for q in range(ROWS_PER_TILE // CHUNK):
        pltpu.sync_copy(rows0, accum_sh.at[pl.ds(base_row + q * CHUNK, CHUNK)])
    pltpu.sync_copy(zb_v, deg_sh.at[pl.ds(base_row, ROWS_PER_TILE)])
    plsc.subcore_barrier()

    chunk0 = wid * N_CHUNKS
    tile_edge0 = wid * EDGES_PER_TILE


    def fetch_idx(b, p, j):
        pltpu.sync_copy(idx_hbm.at[chunk0 + 2 * j + b], ivs[b][p])

    def fire_gather(b, p, j):
        e0 = pl.multiple_of(tile_edge0 + (2 * j + b) * CHUNK, CHUNK)
        pltpu.async_copy(w_hbm.at[pl.ds(e0, CHUNK)], ws[b], sgs[b])
        pltpu.async_copy(h_hbm.at[ivs[b][p].at[1]], rows[b], sgs[b])

    def drain_gather(b):
        pltpu.make_async_copy(w_hbm.at[pl.ds(0, CHUNK)], ws[b], sgs[b]).wait()
        pltpu.make_async_copy(h_hbm.at[pl.ds(0, CHUNK)], rows[b], sgs[b]).wait()

    def fire_scatter(b, p):
        pltpu.async_copy(rows[b], accum_sh.at[ivs[b][p].at[0]], sss[b],
                         add=True)
        pltpu.async_copy(ws[b], deg_sh.at[ivs[b][p].at[0]], sss[b], add=True)

    def drain_scatter(b):
        pltpu.make_async_copy(h_hbm.at[pl.ds(0, CHUNK)], rows[b], sss[b]).wait()
        pltpu.make_async_copy(w_hbm.at[pl.ds(0, CHUNK)], ws[b], sss[b]).wait()

    def scale(b):
        @plsc.parallel_loop(0, CHUNK // 16, unroll=2)
        def _(g):
            w16 = ws[b][pl.ds(g * 16, 16)]
            e0 = g * 16
            for t in range(16):
                wb = jnp.broadcast_to(w16[t], (16,))
                for j in range(8):
                    sl = pl.ds(16 * j, 16)
                    rows[b][e0 + t, sl] = rows[b][e0 + t, sl] * wb

    def body(j, p, prefetch):
        for b in range(2):
            drain_gather(b)
            scale(b)
            fire_scatter(b, p)
        if prefetch:
            for b in range(2):
                fetch_idx(b, 1 - p, j + 1)
                drain_scatter(b)
                fire_gather(b, 1 - p, j + 1)
        else:
            for b in range(2):
                drain_scatter(b)

    n_j = N_CHUNKS // 2

    for b in range(2):
        fetch_idx(b, 0, 0)
        fire_gather(b, 0, 0)

    @pl.loop(0, n_j // 2 - 1)
    def _(i):
        body(2 * i, 0, True)
        body(2 * i + 1, 1, True)

    body(n_j - 2, 0, True)
    body(n_j - 1, 1, False)

    plsc.subcore_barrier()

    pltpu.sync_copy(accum_sh.at[pl.ds(base_row, ROWS_PER_TILE)],
                    outp_hbm.at[c].at[pl.ds(base_row, ROWS_PER_TILE)])
    pltpu.sync_copy(deg_sh.at[pl.ds(base_row, ROWS_PER_TILE)],
                    degp_hbm.at[c].at[pl.ds(base_row, ROWS_PER_TILE)])


def _sparse(h, idx_all, w1d):
    fn = pl.kernel(
        _sc_body,
        out_type=[jax.ShapeDtypeStruct((2, N_PAD, D), jnp.float32),
                  jax.ShapeDtypeStruct((2, N_PAD), jnp.float32)],
        mesh=plsc.VectorSubcoreMesh(core_axis_name="c", subcore_axis_name="s"),
        scratch_types=[
            pltpu.VMEM((2, 128), jnp.int32),
            pltpu.VMEM((2, 128), jnp.int32),
            pltpu.VMEM((2, 128), jnp.int32),
            pltpu.VMEM((2, 128), jnp.int32),
            pltpu.VMEM((CHUNK,), jnp.float32),
            pltpu.VMEM((CHUNK,), jnp.float32),
            pltpu.VMEM((CHUNK, D), jnp.float32),
            pltpu.VMEM((CHUNK, D), jnp.float32),
            pltpu.VMEM((ROWS_PER_TILE,), jnp.float32),
            pltpu.SemaphoreType.DMA,
            pltpu.SemaphoreType.DMA,
            pltpu.SemaphoreType.DMA,
            pltpu.SemaphoreType.DMA,
            pltpu.VMEM_SHARED((N_PAD, D), jnp.float32),
            pltpu.VMEM_SHARED((N_PAD,), jnp.float32),
        ],
    )
    return fn(h, idx_all, w1d)


def _epilogue_body(p0_ref, p1_ref, d0_ref, d1_ref, o_ref):
    d = d0_ref[...] + d1_ref[...]
    d = jnp.where(d < 0.5, d + 1.0, d)
    o_ref[...] = (p0_ref[...] + p1_ref[...]) * (1.0 / d)


def _epilogue(p0, p1, d0, d1):
    n = p0.shape[0]
    return pl.pallas_call(
        _epilogue_body,
        out_shape=jax.ShapeDtypeStruct((n, D), jnp.float32),
    )(p0, p1, d0, d1)


def kernel(x_, edge_index, edge_weight, W, b):
    n = x_.shape[0]
    e = edge_index.shape[1]
    h = _dense(x_, W, b)

    src = edge_index[0]
    dst = edge_index[1]
    pad = E_PAD - e
    pad_idx = (jnp.arange(pad, dtype=jnp.int32) * 37) % n
    src_p = jnp.concatenate([src, pad_idx]).reshape(E_PAD // CHUNK, 1, 128)
    dst_p = jnp.concatenate([dst, pad_idx]).reshape(E_PAD // CHUNK, 1, 128)
    idx_all = jnp.concatenate([src_p, dst_p], axis=1)
    w_p = jnp.concatenate([edge_weight, jnp.zeros((pad,), jnp.float32)])

    outp, degp = _sparse(h, idx_all, w_p)
    return _epilogue(outp[0, :n], outp[1, :n],
                     degp[0, :n, None], degp[1, :n, None])


# speedup vs baseline: 20.2335x; 1.0027x over previous
import jax
import jax.numpy as jnp
from jax import lax
from jax.experimental import pallas as pl
from jax.experimental.pallas import tpu as pltpu
from jax.experimental.pallas import tpu_sc as plsc

D = 128
N_PAD = 10240
ROWS_PER_TILE = 640
E_PAD = 327680
EDGES_PER_TILE = E_PAD // 32
CHUNK = 128
N_CHUNKS = EDGES_PER_TILE // CHUNK


def _dense_body(x_ref, w_ref, b_ref, h_ref):
    acc = lax.dot_general(x_ref[...], w_ref[...], (((1,), (1,)), ((), ())),
                          preferred_element_type=jnp.float32)
    h_ref[...] = jnp.maximum(acc + b_ref[...], 0.0)


def _dense(x, W, b):
    m = x.shape[0]
    bm = 1000
    return pl.pallas_call(
        _dense_body,
        grid=(m // bm,),
        in_specs=[pl.BlockSpec((bm, D), lambda i: (i, 0)),
                  pl.BlockSpec((D, D), lambda i: (0, 0)),
                  pl.BlockSpec((1, D), lambda i: (0, 0))],
        out_specs=pl.BlockSpec((bm, D), lambda i: (i, 0)),
        out_shape=jax.ShapeDtypeStruct((m, D), jnp.float32),
    )(x, W, b.reshape(1, D))


def _sc_body(h_hbm, idx_hbm, w_hbm, outp_hbm, degp_hbm,
             iv00, iv01, iv02, iv03, iv10, iv11, iv12, iv13,
             w0, w1, rows0, rows1, zb_v,
             sg0, sg1, ss0, ss1, si0, si1, accum_sh, deg_sh):
    c = lax.axis_index("c")
    s = lax.axis_index("s")
    wid = c * 16 + s

    ivs = ((iv00, iv01, iv02, iv03),
           (iv10, iv11, iv12, iv13))
    ws = (w0, w1)
    rows = (rows0, rows1)
    sgs = (sg0, sg1)
    sss = (ss0, ss1)
    sis = (si0, si1)

    zeros16 = jnp.zeros((16,), jnp.float32)

    @pl.loop(0, CHUNK)
    def _(r):
        for j in range(8):
            rows0[r, pl.ds(16 * j, 16)] = zeros16

    @pl.loop(0, ROWS_PER_TILE // 16)
    def _(i):
        zb_v[pl.ds(pl.multiple_of(i * 16, 16), 16)] = zeros16

    base_row = pl.multiple_of(s * ROWS_PER_TILE, ROWS_PER_TILE)
    for q in range(ROWS_PER_TILE // CHUNK):
        pltpu.sync_copy(rows0, accum_sh.at[pl.ds(base_row + q * CHUNK, CHUNK)])
    pltpu.sync_copy(zb_v, deg_sh.at[pl.ds(base_row, ROWS_PER_TILE)])
    plsc.subcore_barrier()

    chunk0 = wid * N_CHUNKS
    tile_edge0 = wid * EDGES_PER_TILE


    def fire_idx(b, q, j):
        pltpu.async_copy(idx_hbm.at[chunk0 + 2 * j + b], ivs[b][q], sis[b])

    def drain_idx(b, q):
        pltpu.make_async_copy(idx_hbm.at[chunk0], ivs[b][q], sis[b]).wait()

    def fire_gather(b, q, j):
        e0 = pl.multiple_of(tile_edge0 + (2 * j + b) * CHUNK, CHUNK)
        pltpu.async_copy(w_hbm.at[pl.ds(e0, CHUNK)], ws[b], sgs[b])
        pltpu.async_copy(h_hbm.at[ivs[b][q].at[1]], rows[b], sgs[b])

    def drain_gather(b):
        pltpu.make_async_copy(w_hbm.at[pl.ds(0, CHUNK)], ws[b], sgs[b]).wait()
        pltpu.make_async_copy(h_hbm.at[pl.ds(0, CHUNK)], rows[b], sgs[b]).wait()

    def fire_scatter(b, q):
        pltpu.async_copy(rows[b], accum_sh.at[ivs[b][q].at[0]], sss[b],
                         add=True)
        pltpu.async_copy(ws[b], deg_sh.at[ivs[b][q].at[0]], sss[b], add=True)

    def drain_scatter(b):
        pltpu.make_async_copy(h_hbm.at[pl.ds(0, CHUNK)], rows[b], sss[b]).wait()
        pltpu.make_async_copy(w_hbm.at[pl.ds(0, CHUNK)], ws[b], sss[b]).wait()

    def scale(b):
        @plsc.parallel_loop(0, CHUNK // 16, unroll=2)
        def _(g):
            w16 = ws[b][pl.ds(g * 16, 16)]
            e0 = g * 16
            for t in range(16):
                wb = jnp.broadcast_to(w16[t], (16,))
                for j in range(8):
                    sl = pl.ds(16 * j, 16)
                    rows[b][e0 + t, sl] = rows[b][e0 + t, sl] * wb

    def body(j, q, idx_pf, gather_pf):
        for b in range(2):
            drain_gather(b)
            scale(b)
            fire_scatter(b, q)
        for b in range(2):
            if idx_pf:
                fire_idx(b, (q + 2) % 4, j + 2)
            drain_scatter(b)
            if gather_pf:
                drain_idx(b, (q + 1) % 4)
                fire_gather(b, (q + 1) % 4, j + 1)

    n_j = N_CHUNKS // 2

    for b in range(2):
        fire_idx(b, 0, 0)
    for b in range(2):
        drain_idx(b, 0)
        fire_gather(b, 0, 0)
        fire_idx(b, 1, 1)

    @pl.loop(0, n_j // 4 - 1)
    def _(i):
        for t in range(4):
            body(4 * i + t, t, True, True)

    body(n_j - 4, 0, True, True)
    body(n_j - 3, 1, True, True)
    body(n_j - 2, 2, False, True)
    body(n_j - 1, 3, False, False)

    plsc.subcore_barrier()

    pltpu.sync_copy(accum_sh.at[pl.ds(base_row, ROWS_PER_TILE)],
                    outp_hbm.at[c].at[pl.ds(base_row, ROWS_PER_TILE)])
    pltpu.sync_copy(deg_sh.at[pl.ds(base_row, ROWS_PER_TILE)],
                    degp_hbm.at[c].at[pl.ds(base_row, ROWS_PER_TILE)])


def _sparse(h, idx_all, w1d):
    fn = pl.kernel(
        _sc_body,
        out_type=[jax.ShapeDtypeStruct((2, N_PAD, D), jnp.float32),
                  jax.ShapeDtypeStruct((2, N_PAD), jnp.float32)],
        mesh=plsc.VectorSubcoreMesh(core_axis_name="c", subcore_axis_name="s"),
        scratch_types=[
            pltpu.VMEM((2, 128), jnp.int32),
            pltpu.VMEM((2, 128), jnp.int32),
            pltpu.VMEM((2, 128), jnp.int32),
            pltpu.VMEM((2, 128), jnp.int32),
            pltpu.VMEM((2, 128), jnp.int32),
            pltpu.VMEM((2, 128), jnp.int32),
            pltpu.VMEM((2, 128), jnp.int32),
            pltpu.VMEM((2, 128), jnp.int32),
            pltpu.VMEM((CHUNK,), jnp.float32),
            pltpu.VMEM((CHUNK,), jnp.float32),
            pltpu.VMEM((CHUNK, D), jnp.float32),
            pltpu.VMEM((CHUNK, D), jnp.float32),
            pltpu.VMEM((ROWS_PER_TILE,), jnp.float32),
            pltpu.SemaphoreType.DMA,
            pltpu.SemaphoreType.DMA,
            pltpu.SemaphoreType.DMA,
            pltpu.SemaphoreType.DMA,
            pltpu.SemaphoreType.DMA,
            pltpu.SemaphoreType.DMA,
            pltpu.VMEM_SHARED((N_PAD, D), jnp.float32),
            pltpu.VMEM_SHARED((N_PAD,), jnp.float32),
        ],
    )
    return fn(h, idx_all, w1d)


def _epilogue_body(p0_ref, p1_ref, d0_ref, d1_ref, o_ref):
    d = d0_ref[...] + d1_ref[...]
    d = jnp.where(d < 0.5, d + 1.0, d)
    o_ref[...] = (p0_ref[...] + p1_ref[...]) * (1.0 / d)


def _epilogue(p0, p1, d0, d1):
    n = p0.shape[0]
    bm = 1000
    return pl.pallas_call(
        _epilogue_body,
        grid=(n // bm,),
        in_specs=[pl.BlockSpec((bm, D), lambda i: (i, 0)),
                  pl.BlockSpec((bm, D), lambda i: (i, 0)),
                  pl.BlockSpec((bm, 1), lambda i: (i, 0)),
                  pl.BlockSpec((bm, 1), lambda i: (i, 0))],
        out_specs=pl.BlockSpec((bm, D), lambda i: (i, 0)),
        out_shape=jax.ShapeDtypeStruct((n, D), jnp.float32),
    )(p0, p1, d0, d1)


def kernel(x_, edge_index, edge_weight, W, b):
    n = x_.shape[0]
    e = edge_index.shape[1]
    h = _dense(x_, W, b)

    src = edge_index[0]
    dst = edge_index[1]
    pad = E_PAD - e
    pad_idx = (jnp.arange(pad, dtype=jnp.int32) * 37) % n
    src_p = jnp.concatenate([src, pad_idx]).reshape(E_PAD // CHUNK, 1, 128)
    dst_p = jnp.concatenate([dst, pad_idx]).reshape(E_PAD // CHUNK, 1, 128)
    idx_all = jnp.concatenate([src_p, dst_p], axis=1)
    w_p = jnp.concatenate([edge_weight, jnp.zeros((pad,), jnp.float32)])

    outp, degp = _sparse(h, idx_all, w_p)
    return _epilogue(outp[0, :n], outp[1, :n],
                     degp[0, :n, None], degp[1, :n, None])


# speedup vs baseline: 21.2946x; 1.0524x over previous
import jax
import jax.numpy as jnp
from jax import lax
from jax.experimental import pallas as pl
from jax.experimental.pallas import tpu as pltpu
from jax.experimental.pallas import tpu_sc as plsc

D = 128
N_PAD = 10240
ROWS_PER_TILE = 640
E_PAD = 327680
EDGES_PER_TILE = E_PAD // 32
CHUNK = 128
N_CHUNKS = EDGES_PER_TILE // CHUNK


def _dense_body(x_ref, w_ref, b_ref, h_ref):
    acc = lax.dot_general(x_ref[...], w_ref[...], (((1,), (1,)), ((), ())),
                          preferred_element_type=jnp.float32)
    h_ref[...] = jnp.maximum(acc + b_ref[...], 0.0)


def _dense(x, W, b):
    m = x.shape[0]
    bm = 1000
    return pl.pallas_call(
        _dense_body,
        grid=(m // bm,),
        in_specs=[pl.BlockSpec((bm, D), lambda i: (i, 0)),
                  pl.BlockSpec((D, D), lambda i: (0, 0)),
                  pl.BlockSpec((1, D), lambda i: (0, 0))],
        out_specs=pl.BlockSpec((bm, D), lambda i: (i, 0)),
        out_shape=jax.ShapeDtypeStruct((m, D), jnp.float32),
    )(x, W, b.reshape(1, D))


def _sc_body(h_hbm, idx_hbm, w_hbm, outp_hbm, degp_hbm,
             iv00, iv01, iv02, iv03, iv10, iv11, iv12, iv13,
             w0, w1, rows0, rows1, zb_v,
             sg0, sg1, ss0, ss1, si0, si1, accum_sh, deg_sh):
    c = lax.axis_index("c")
    s = lax.axis_index("s")
    wid = c * 16 + s

    ivs = ((iv00, iv01, iv02, iv03),
           (iv10, iv11, iv12, iv13))
    ws = (w0, w1)
    rows = (rows0, rows1)
    sgs = (sg0, sg1)
    sss = (ss0, ss1)
    sis = (si0, si1)

    zeros16 = jnp.zeros((16,), jnp.float32)

    @pl.loop(0, CHUNK)
    def _(r):
        for j in range(8):
            rows0[r, pl.ds(16 * j, 16)] = zeros16

    @pl.loop(0, ROWS_PER_TILE // 16)
    def _(i):
        zb_v[pl.ds(pl.multiple_of(i * 16, 16), 16)] = zeros16

    base_row = pl.multiple_of(s * ROWS_PER_TILE, ROWS_PER_TILE)
    for q in range(ROWS_PER_TILE // CHUNK):
        pltpu.sync_copy(rows0, accum_sh.at[pl.ds(base_row + q * CHUNK, CHUNK)])
    pltpu.sync_copy(zb_v, deg_sh.at[pl.ds(base_row, ROWS_PER_TILE)])
    plsc.subcore_barrier()

    chunk0 = wid * N_CHUNKS
    tile_edge0 = wid * EDGES_PER_TILE


    def fire_idx(b, q, j):
        pltpu.async_copy(idx_hbm.at[chunk0 + 2 * j + b], ivs[b][q], sis[b])

    def drain_idx(b, q):
        pltpu.make_async_copy(idx_hbm.at[chunk0], ivs[b][q], sis[b]).wait()

    def fire_gather(b, q, j):
        e0 = pl.multiple_of(tile_edge0 + (2 * j + b) * CHUNK, CHUNK)
        pltpu.async_copy(w_hbm.at[pl.ds(e0, CHUNK)], ws[b], sgs[b])
        pltpu.async_copy(h_hbm.at[ivs[b][q].at[1]], rows[b], sgs[b])

    def drain_gather(b):
        pltpu.make_async_copy(w_hbm.at[pl.ds(0, CHUNK)], ws[b], sgs[b]).wait()
        pltpu.make_async_copy(h_hbm.at[pl.ds(0, CHUNK)], rows[b], sgs[b]).wait()

    def fire_scatter(b, q):
        pltpu.async_copy(rows[b], accum_sh.at[ivs[b][q].at[0]], sss[b],
                         add=True)
        pltpu.async_copy(ws[b], deg_sh.at[ivs[b][q].at[0]], sss[b], add=True)

    def drain_scatter(b):
        pltpu.make_async_copy(h_hbm.at[pl.ds(0, CHUNK)], rows[b], sss[b]).wait()
        pltpu.make_async_copy(w_hbm.at[pl.ds(0, CHUNK)], ws[b], sss[b]).wait()

    def scale(b):
        @plsc.parallel_loop(0, CHUNK // 16, unroll=2)
        def _(g):
            w16 = ws[b][pl.ds(g * 16, 16)]
            e0 = g * 16
            for t in range(16):
                wb = jnp.broadcast_to(w16[t], (16,))
                for j in range(8):
                    sl = pl.ds(16 * j, 16)
                    rows[b][e0 + t, sl] = rows[b][e0 + t, sl] * wb

    def body(j, q, idx_pf, gather_pf):
        for b in range(2):
            drain_gather(b)
            scale(b)
            fire_scatter(b, q)
            if idx_pf:
                fire_idx(b, (q + 2) % 4, j + 2)
            drain_scatter(b)
            if gather_pf:
                drain_idx(b, (q + 1) % 4)
                fire_gather(b, (q + 1) % 4, j + 1)

    n_j = N_CHUNKS // 2

    for b in range(2):
        fire_idx(b, 0, 0)
    for b in range(2):
        drain_idx(b, 0)
        fire_gather(b, 0, 0)
        fire_idx(b, 1, 1)

    @pl.loop(0, n_j // 4 - 1)
    def _(i):
        for t in range(4):
            body(4 * i + t, t, True, True)

    body(n_j - 4, 0, True, True)
    body(n_j - 3, 1, True, True)
    body(n_j - 2, 2, False, True)
    body(n_j - 1, 3, False, False)

    plsc.subcore_barrier()

    pltpu.sync_copy(accum_sh.at[pl.ds(base_row, ROWS_PER_TILE)],
                    outp_hbm.at[c].at[pl.ds(base_row, ROWS_PER_TILE)])
    pltpu.sync_copy(deg_sh.at[pl.ds(base_row, ROWS_PER_TILE)],
                    degp_hbm.at[c].at[pl.ds(base_row, ROWS_PER_TILE)])


def _sparse(h, idx_all, w1d):
    fn = pl.kernel(
        _sc_body,
        out_type=[jax.ShapeDtypeStruct((2, N_PAD, D), jnp.float32),
                  jax.ShapeDtypeStruct((2, N_PAD), jnp.float32)],
        mesh=plsc.VectorSubcoreMesh(core_axis_name="c", subcore_axis_name="s"),
        scratch_types=[
            pltpu.VMEM((2, 128), jnp.int32),
            pltpu.VMEM((2, 128), jnp.int32),
            pltpu.VMEM((2, 128), jnp.int32),
            pltpu.VMEM((2, 128), jnp.int32),
            pltpu.VMEM((2, 128), jnp.int32),
            pltpu.VMEM((2, 128), jnp.int32),
            pltpu.VMEM((2, 128), jnp.int32),
            pltpu.VMEM((2, 128), jnp.int32),
            pltpu.VMEM((CHUNK,), jnp.float32),
            pltpu.VMEM((CHUNK,), jnp.float32),
            pltpu.VMEM((CHUNK, D), jnp.float32),
            pltpu.VMEM((CHUNK, D), jnp.float32),
            pltpu.VMEM((ROWS_PER_TILE,), jnp.float32),
            pltpu.SemaphoreType.DMA,
            pltpu.SemaphoreType.DMA,
            pltpu.SemaphoreType.DMA,
            pltpu.SemaphoreType.DMA,
            pltpu.SemaphoreType.DMA,
            pltpu.SemaphoreType.DMA,
            pltpu.VMEM_SHARED((N_PAD, D), jnp.float32),
            pltpu.VMEM_SHARED((N_PAD,), jnp.float32),
        ],
    )
    return fn(h, idx_all, w1d)


def _epilogue_body(p0_ref, p1_ref, d0_ref, d1_ref, o_ref):
    d = d0_ref[...] + d1_ref[...]
    d = jnp.where(d < 0.5, d + 1.0, d)
    o_ref[...] = (p0_ref[...] + p1_ref[...]) * (1.0 / d)


def _epilogue(p0, p1, d0, d1):
    n = p0.shape[0]
    bm = 1000
    return pl.pallas_call(
        _epilogue_body,
        grid=(n // bm,),
        in_specs=[pl.BlockSpec((bm, D), lambda i: (i, 0)),
                  pl.BlockSpec((bm, D), lambda i: (i, 0)),
                  pl.BlockSpec((bm, 1), lambda i: (i, 0)),
                  pl.BlockSpec((bm, 1), lambda i: (i, 0))],
        out_specs=pl.BlockSpec((bm, D), lambda i: (i, 0)),
        out_shape=jax.ShapeDtypeStruct((n, D), jnp.float32),
    )(p0, p1, d0, d1)


def kernel(x_, edge_index, edge_weight, W, b):
    n = x_.shape[0]
    e = edge_index.shape[1]
    h = _dense(x_, W, b)

    src = edge_index[0]
    dst = edge_index[1]
    pad = E_PAD - e
    pad_idx = (jnp.arange(pad, dtype=jnp.int32) * 37) % n
    src_p = jnp.concatenate([src, pad_idx]).reshape(E_PAD // CHUNK, 1, 128)
    dst_p = jnp.concatenate([dst, pad_idx]).reshape(E_PAD // CHUNK, 1, 128)
    idx_all = jnp.concatenate([src_p, dst_p], axis=1)
    w_p = jnp.concatenate([edge_weight, jnp.zeros((pad,), jnp.float32)])

    outp, degp = _sparse(h, idx_all, w_p)
    return _epilogue(outp[0, :n], outp[1, :n],
                     degp[0, :n, None], degp[1, :n, None])


# speedup vs baseline: 21.5241x; 1.0108x over previous
import jax
import jax.numpy as jnp
from jax import lax
from jax.experimental import pallas as pl
from jax.experimental.pallas import tpu as pltpu
from jax.experimental.pallas import tpu_sc as plsc

D = 128
N_PAD = 10240
ROWS_PER_TILE = 640
E_PAD = 327680
EDGES_PER_TILE = E_PAD // 32
CHUNK = 128
N_CHUNKS = EDGES_PER_TILE // CHUNK


def _dense_body(x_ref, w_ref, b_ref, h_ref):
    acc = lax.dot_general(x_ref[...], w_ref[...], (((1,), (1,)), ((), ())),
                          preferred_element_type=jnp.float32)
    h_ref[...] = jnp.maximum(acc + b_ref[...], 0.0)


def _dense(x, W, b):
    m = x.shape[0]
    bm = 1000
    return pl.pallas_call(
        _dense_body,
        grid=(m // bm,),
        in_specs=[pl.BlockSpec((bm, D), lambda i: (i, 0)),
                  pl.BlockSpec((D, D), lambda i: (0, 0)),
                  pl.BlockSpec((1, D), lambda i: (0, 0))],
        out_specs=pl.BlockSpec((bm, D), lambda i: (i, 0)),
        out_shape=jax.ShapeDtypeStruct((m, D), jnp.float32),
    )(x, W, b.reshape(1, D))


def _sc_body(h_hbm, idx_hbm, w_hbm, outp_hbm, degp_hbm,
             iv00, iv01, iv02, iv03, iv10, iv11, iv12, iv13,
             w0, w1, rows0, rows1, zb_v,
             sg0, sg1, ss0, ss1, si0, si1, accum_sh, deg_sh):
    c = lax.axis_index("c")
    s = lax.axis_index("s")
    wid = c * 16 + s

    ivs = ((iv00, iv01, iv02, iv03),
           (iv10, iv11, iv12, iv13))
    ws = (w0, w1)
    rows = (rows0, rows1)
    sgs = (sg0, sg1)
    sss = (ss0, ss1)
    sis = (si0, si1)

    zeros16 = jnp.zeros((16,), jnp.float32)

    @pl.loop(0, CHUNK)
    def _(r):
        for j in range(8):
            rows0[r, pl.ds(16 * j, 16)] = zeros16

    @pl.loop(0, ROWS_PER_TILE // 16)
    def _(i):
        zb_v[pl.ds(pl.multiple_of(i * 16, 16), 16)] = zeros16

    base_row = pl.multiple_of(s * ROWS_PER_TILE, ROWS_PER_TILE)
    for q in range(ROWS_PER_TILE // CHUNK):
        pltpu.sync_copy(rows0, accum_sh.at[pl.ds(base_row + q * CHUNK, CHUNK)])
    pltpu.sync_copy(zb_v, deg_sh.at[pl.ds(base_row, ROWS_PER_TILE)])
    plsc.subcore_barrier()

    chunk0 = wid * N_CHUNKS
    tile_edge0 = wid * EDGES_PER_TILE


    def fire_idx(b, q, j):
        pltpu.async_copy(idx_hbm.at[chunk0 + 2 * j + b], ivs[b][q], sis[b])

    def drain_idx(b, q):
        pltpu.make_async_copy(idx_hbm.at[chunk0], ivs[b][q], sis[b]).wait()

    def fire_gather(b, q, j):
        pltpu.async_copy(h_hbm.at[ivs[b][q].at[1]], rows[b], sgs[b])

    def drain_gather(b):
        pltpu.make_async_copy(h_hbm.at[pl.ds(0, CHUNK)], rows[b], sgs[b]).wait()

    def fire_scatter(b, q):
        pltpu.async_copy(rows[b], accum_sh.at[ivs[b][q].at[0]], sss[b],
                         add=True)
        pltpu.async_copy(ws[b], deg_sh.at[ivs[b][q].at[0]], sss[b], add=True)

    def drain_scatter(b):
        pltpu.make_async_copy(h_hbm.at[pl.ds(0, CHUNK)], rows[b], sss[b]).wait()
        pltpu.make_async_copy(w_hbm.at[pl.ds(0, CHUNK)], ws[b], sss[b]).wait()

    def scale(b, q):
        @plsc.parallel_loop(0, CHUNK // 16, unroll=2)
        def _(g):
            w16 = lax.bitcast_convert_type(ivs[b][q][2, pl.ds(g * 16, 16)],
                                           jnp.float32)
            ws[b][pl.ds(g * 16, 16)] = w16
            e0 = g * 16
            for t in range(16):
                wb = jnp.broadcast_to(w16[t], (16,))
                for j in range(8):
                    sl = pl.ds(16 * j, 16)
                    rows[b][e0 + t, sl] = rows[b][e0 + t, sl] * wb

    def body(j, q, idx_pf, gather_pf):
        for b in range(2):
            drain_gather(b)
            scale(b, q)
            fire_scatter(b, q)
            if idx_pf:
                fire_idx(b, (q + 2) % 4, j + 2)
            drain_scatter(b)
            if gather_pf:
                drain_idx(b, (q + 1) % 4)
                fire_gather(b, (q + 1) % 4, j + 1)

    n_j = N_CHUNKS // 2

    for b in range(2):
        fire_idx(b, 0, 0)
    for b in range(2):
        drain_idx(b, 0)
        fire_gather(b, 0, 0)
        fire_idx(b, 1, 1)

    @pl.loop(0, n_j // 4 - 1)
    def _(i):
        for t in range(4):
            body(4 * i + t, t, True, True)

    body(n_j - 4, 0, True, True)
    body(n_j - 3, 1, True, True)
    body(n_j - 2, 2, False, True)
    body(n_j - 1, 3, False, False)

    plsc.subcore_barrier()

    pltpu.sync_copy(accum_sh.at[pl.ds(base_row, ROWS_PER_TILE)],
                    outp_hbm.at[c].at[pl.ds(base_row, ROWS_PER_TILE)])
    pltpu.sync_copy(deg_sh.at[pl.ds(base_row, ROWS_PER_TILE)],
                    degp_hbm.at[c].at[pl.ds(base_row, ROWS_PER_TILE)])


def _sparse(h, idx_all, w1d):
    fn = pl.kernel(
        _sc_body,
        out_type=[jax.ShapeDtypeStruct((2, N_PAD, D), jnp.float32),
                  jax.ShapeDtypeStruct((2, N_PAD), jnp.float32)],
        mesh=plsc.VectorSubcoreMesh(core_axis_name="c", subcore_axis_name="s"),
        scratch_types=[
            pltpu.VMEM((3, 128), jnp.int32),
            pltpu.VMEM((3, 128), jnp.int32),
            pltpu.VMEM((3, 128), jnp.int32),
            pltpu.VMEM((3, 128), jnp.int32),
            pltpu.VMEM((3, 128), jnp.int32),
            pltpu.VMEM((3, 128), jnp.int32),
            pltpu.VMEM((3, 128), jnp.int32),
            pltpu.VMEM((3, 128), jnp.int32),
            pltpu.VMEM((CHUNK,), jnp.float32),
            pltpu.VMEM((CHUNK,), jnp.float32),
            pltpu.VMEM((CHUNK, D), jnp.float32),
            pltpu.VMEM((CHUNK, D), jnp.float32),
            pltpu.VMEM((ROWS_PER_TILE,), jnp.float32),
            pltpu.SemaphoreType.DMA,
            pltpu.SemaphoreType.DMA,
            pltpu.SemaphoreType.DMA,
            pltpu.SemaphoreType.DMA,
            pltpu.SemaphoreType.DMA,
            pltpu.SemaphoreType.DMA,
            pltpu.VMEM_SHARED((N_PAD, D), jnp.float32),
            pltpu.VMEM_SHARED((N_PAD,), jnp.float32),
        ],
    )
    return fn(h, idx_all, w1d)


def _epilogue_body(p0_ref, p1_ref, d0_ref, d1_ref, o_ref):
    d = d0_ref[...] + d1_ref[...]
    d = jnp.where(d < 0.5, d + 1.0, d)
    o_ref[...] = (p0_ref[...] + p1_ref[...]) * (1.0 / d)


def _epilogue(p0, p1, d0, d1):
    n = p0.shape[0]
    bm = 1000
    return pl.pallas_call(
        _epilogue_body,
        grid=(n // bm,),
        in_specs=[pl.BlockSpec((bm, D), lambda i: (i, 0)),
                  pl.BlockSpec((bm, D), lambda i: (i, 0)),
                  pl.BlockSpec((bm, 1), lambda i: (i, 0)),
                  pl.BlockSpec((bm, 1), lambda i: (i, 0))],
        out_specs=pl.BlockSpec((bm, D), lambda i: (i, 0)),
        out_shape=jax.ShapeDtypeStruct((n, D), jnp.float32),
    )(p0, p1, d0, d1)


def kernel(x_, edge_index, edge_weight, W, b):
    n = x_.shape[0]
    e = edge_index.shape[1]
    h = _dense(x_, W, b)

    src = edge_index[0]
    dst = edge_index[1]
    pad = E_PAD - e
    pad_idx = (jnp.arange(pad, dtype=jnp.int32) * 37) % n
    src_p = jnp.concatenate([src, pad_idx]).reshape(E_PAD // CHUNK, 1, 128)
    dst_p = jnp.concatenate([dst, pad_idx]).reshape(E_PAD // CHUNK, 1, 128)
    w_p = jnp.concatenate([edge_weight, jnp.zeros((pad,), jnp.float32)])
    w_bits = lax.bitcast_convert_type(w_p, jnp.int32)
    w_bits = w_bits.reshape(E_PAD // CHUNK, 1, 128)
    idx_all = jnp.concatenate([src_p, dst_p, w_bits], axis=1)

    outp, degp = _sparse(h, idx_all, w_p)
    return _epilogue(outp[0, :n], outp[1, :n],
                     degp[0, :n, None], degp[1, :n, None])
